# Initial kernel scaffold; baseline (speedup 1.0000x reference)
import functools

import jax
import jax.numpy as jnp
from jax import lax
from jax.experimental import pallas as pl
from jax.experimental.pallas import tpu as pltpu
from jax.experimental.pallas import tpu_sc as plsc

N = 10000
M = 5000
E = 320000
D = 128

M_PAD = 5008
FPT = 4
CHUNK = 8000
NCHUNK = E // CHUNK
VPC = CHUNK // 16

NEG_INF = float("-inf")


def _u_body(x_ref, pos_ref, w_ref, u_ref):
    xw = jnp.dot(x_ref[...], w_ref[:D, :], preferred_element_type=jnp.float32)
    pw = jnp.dot(pos_ref[...], w_ref[D:, :], preferred_element_type=jnp.float32)
    u_ref[...] = xw + pw


def _c_body(op_ref, w_ref, b_ref, c_ref):
    pw = jnp.dot(op_ref[...], w_ref[D:, :], preferred_element_type=jnp.float32)
    c_ref[...] = b_ref[...][None, :] - pw


def _compute_u(x, pos, W1):
    bn = 1000
    return pl.pallas_call(
        _u_body,
        grid=(N // bn,),
        in_specs=[
            pl.BlockSpec((bn, D), lambda i: (i, 0)),
            pl.BlockSpec((bn, 3), lambda i: (i, 0)),
            pl.BlockSpec((D + 3, D), lambda i: (0, 0)),
        ],
        out_specs=pl.BlockSpec((bn, D), lambda i: (i, 0)),
        out_shape=jax.ShapeDtypeStruct((N, D), jnp.float32),
    )(x, pos, W1)


def _compute_c(out_pos, W1, b1):
    bm = 1000
    return pl.pallas_call(
        _c_body,
        grid=(M // bm,),
        in_specs=[
            pl.BlockSpec((bm, 3), lambda i: (i, 0)),
            pl.BlockSpec((D + 3, D), lambda i: (0, 0)),
            pl.BlockSpec((D,), lambda i: (0,)),
        ],
        out_specs=pl.BlockSpec((bm, D), lambda i: (i, 0)),
        out_shape=jax.ShapeDtypeStruct((M, D), jnp.float32),
    )(out_pos, W1, b1)


def _sc_body(uT, cT, src_hbm, dst_hbm, outT,
             u0, u1, u2, u3, a0, a1, a2, a3, c2d, sbuf, dbuf):
    us = (u0, u1, u2, u3)
    accs = (a0, a1, a2, a3)
    wid = lax.axis_index("s") * 2 + lax.axis_index("c")
    row0 = wid * FPT

    for j, u_j in enumerate(us):
        pltpu.sync_copy(uT.at[row0 + j], u_j)
    pltpu.sync_copy(cT.at[pl.ds(row0, FPT), :], c2d)

    minus_inf = jnp.full((16,), NEG_INF, jnp.float32)

    def init_body(i, _):
        for a_j in accs:
            a_j[pl.ds(i * 16, 16)] = minus_inf
        return 0

    lax.fori_loop(0, M_PAD // 16, init_body, 0)

    def chunk_body(k, _):
        pltpu.sync_copy(src_hbm.at[pl.ds(k * CHUNK, CHUNK)], sbuf)
        pltpu.sync_copy(dst_hbm.at[pl.ds(k * CHUNK, CHUNK)], dbuf)

        def vec_body(i, _):
            sv = sbuf[pl.ds(i * 16, 16)]
            dv = dbuf[pl.ds(i * 16, 16)]
            ms = []
            chks = []
            for j in range(FPT):
                g = plsc.load_gather(us[j], [sv])
                o = plsc.load_gather(accs[j], [dv])
                m = jnp.maximum(g, o)
                plsc.store_scatter(accs[j], [dv], m)
                ms.append(m)
                chks.append(plsc.load_gather(accs[j], [dv]))

            def lost(chk):
                return [chk[j] < ms[j] for j in range(FPT)]

            def w_cond(chk):
                n = lost(chk)
                return jnp.any(n[0] | n[1] | n[2] | n[3])

            def w_body(chk):
                n = lost(chk)
                new = []
                for j in range(FPT):
                    plsc.store_scatter(accs[j], [dv], ms[j], mask=n[j])
                    new.append(plsc.load_gather(accs[j], [dv]))
                return tuple(new)

            lax.while_loop(w_cond, w_body, tuple(chks))
            return 0

        lax.fori_loop(0, VPC, vec_body, 0)
        return 0

    lax.fori_loop(0, NCHUNK, chunk_body, 0)

    def ep_body(i, _):
        sl = pl.ds(i * 16, 16)
        for j in range(FPT):
            c2d[j, sl] = jnp.maximum(accs[j][sl] + c2d[j, sl], 0.0)
        return 0

    lax.fori_loop(0, M_PAD // 16, ep_body, 0)
    pltpu.sync_copy(c2d, outT.at[pl.ds(row0, FPT), :])


def _sc_scatter_max(uT, cT, src, dst):
    mesh = plsc.VectorSubcoreMesh(core_axis_name="c", subcore_axis_name="s")
    kfn = functools.partial(
        pl.kernel,
        mesh=mesh,
        out_type=jax.ShapeDtypeStruct((D, M_PAD), jnp.float32),
        scratch_types=[
            pltpu.VMEM((N,), jnp.float32),
            pltpu.VMEM((N,), jnp.float32),
            pltpu.VMEM((N,), jnp.float32),
            pltpu.VMEM((N,), jnp.float32),
            pltpu.VMEM((M_PAD,), jnp.float32),
            pltpu.VMEM((M_PAD,), jnp.float32),
            pltpu.VMEM((M_PAD,), jnp.float32),
            pltpu.VMEM((M_PAD,), jnp.float32),
            pltpu.VMEM((FPT, M_PAD), jnp.float32),
            pltpu.VMEM((CHUNK,), jnp.int32),
            pltpu.VMEM((CHUNK,), jnp.int32),
        ],
    )(_sc_body)
    return kfn(uT, cT, src, dst)


def kernel(x, pos, out_pos, edge_index, W1, b1):
    u = _compute_u(x, pos, W1)
    c = _compute_c(out_pos, W1, b1)
    uT = u.T
    cT = jnp.pad(c, ((0, M_PAD - M), (0, 0))).T
    src = edge_index[0]
    dst = edge_index[1]
    outT = _sc_scatter_max(uT, cT, src, dst)
    agg = outT[:, :M].T
    return (agg, out_pos)


# speedup vs baseline: 2.8376x; 2.8376x over previous
import functools

import jax
import jax.numpy as jnp
from jax import lax
from jax.experimental import pallas as pl
from jax.experimental.pallas import tpu as pltpu
from jax.experimental.pallas import tpu_sc as plsc

N = 10000
M = 5000
E = 320000
D = 128

M_PAD = 5008
FPT = 4
CHUNK = 8000
NCHUNK = E // CHUNK
VPC = CHUNK // 16

NEG_INF = float("-inf")


def _u_body(x_ref, pos_ref, w_ref, u_ref):
    xw = jnp.dot(x_ref[...], w_ref[:D, :], preferred_element_type=jnp.float32)
    pw = jnp.dot(pos_ref[...], w_ref[D:, :], preferred_element_type=jnp.float32)
    u_ref[...] = xw + pw


def _c_body(op_ref, w_ref, b_ref, c_ref):
    pw = jnp.dot(op_ref[...], w_ref[D:, :], preferred_element_type=jnp.float32)
    c_ref[...] = b_ref[...][None, :] - pw


def _compute_u(x, pos, W1):
    bn = 1000
    return pl.pallas_call(
        _u_body,
        grid=(N // bn,),
        in_specs=[
            pl.BlockSpec((bn, D), lambda i: (i, 0)),
            pl.BlockSpec((bn, 3), lambda i: (i, 0)),
            pl.BlockSpec((D + 3, D), lambda i: (0, 0)),
        ],
        out_specs=pl.BlockSpec((bn, D), lambda i: (i, 0)),
        out_shape=jax.ShapeDtypeStruct((N, D), jnp.float32),
    )(x, pos, W1)


def _compute_c(out_pos, W1, b1):
    bm = 1000
    return pl.pallas_call(
        _c_body,
        grid=(M // bm,),
        in_specs=[
            pl.BlockSpec((bm, 3), lambda i: (i, 0)),
            pl.BlockSpec((D + 3, D), lambda i: (0, 0)),
            pl.BlockSpec((D,), lambda i: (0,)),
        ],
        out_specs=pl.BlockSpec((bm, D), lambda i: (i, 0)),
        out_shape=jax.ShapeDtypeStruct((M, D), jnp.float32),
    )(out_pos, W1, b1)


def _sc_body(uT, cT, src_hbm, dst_hbm, outT,
             u0, u1, u2, u3, a0, a1, a2, a3, c2d, sbuf, dbuf):
    us = (u0, u1, u2, u3)
    accs = (a0, a1, a2, a3)
    wid = lax.axis_index("s") * 2 + lax.axis_index("c")
    row0 = wid * FPT

    for j, u_j in enumerate(us):
        pltpu.sync_copy(uT.at[row0 + j], u_j)
    pltpu.sync_copy(cT.at[pl.ds(row0, FPT), :], c2d)

    minus_inf = jnp.full((16,), NEG_INF, jnp.float32)

    def init_body(i, _):
        for a_j in accs:
            a_j[pl.ds(i * 16, 16)] = minus_inf
        return 0

    lax.fori_loop(0, M_PAD // 16, init_body, 0)

    def chunk_body(k, _):
        pltpu.sync_copy(src_hbm.at[pl.ds(k * CHUNK, CHUNK)], sbuf)
        pltpu.sync_copy(dst_hbm.at[pl.ds(k * CHUNK, CHUNK)], dbuf)

        def vec_body(i, _):
            sv = sbuf[pl.ds(i * 16, 16)]
            dv = dbuf[pl.ds(i * 16, 16)]
            ms = []
            chks = []
            for j in range(FPT):
                g = plsc.load_gather(us[j], [sv])
                o = plsc.load_gather(accs[j], [dv])
                m = jnp.maximum(g, o)
                plsc.store_scatter(accs[j], [dv], m)
                ms.append(m)
                chks.append(plsc.load_gather(accs[j], [dv]))

            def lost(chk):
                return [chk[j] < ms[j] for j in range(FPT)]

            def w_cond(chk):
                n = lost(chk)
                return jnp.any(n[0] | n[1] | n[2] | n[3])

            def w_body(chk):
                n = lost(chk)
                new = []
                for j in range(FPT):
                    plsc.store_scatter(accs[j], [dv], ms[j], mask=n[j])
                    new.append(plsc.load_gather(accs[j], [dv]))
                return tuple(new)

            lax.while_loop(w_cond, w_body, tuple(chks))
            return 0

        lax.fori_loop(0, VPC, vec_body, 0)
        return 0

    lax.fori_loop(0, NCHUNK, chunk_body, 0)

    def ep_body(i, _):
        sl = pl.ds(i * 16, 16)
        for j in range(FPT):
            c2d[j, sl] = jnp.maximum(accs[j][sl] + c2d[j, sl], 0.0)
        return 0

    lax.fori_loop(0, M_PAD // 16, ep_body, 0)
    pltpu.sync_copy(c2d, outT.at[pl.ds(row0, FPT), :])


def _sc_scatter_max(uT, cT, src, dst):
    mesh = plsc.VectorSubcoreMesh(core_axis_name="c", subcore_axis_name="s")
    kfn = functools.partial(
        pl.kernel,
        mesh=mesh,
        compiler_params=pltpu.CompilerParams(needs_layout_passes=False),
        out_type=jax.ShapeDtypeStruct((D, M_PAD), jnp.float32),
        scratch_types=[
            pltpu.VMEM((N,), jnp.float32),
            pltpu.VMEM((N,), jnp.float32),
            pltpu.VMEM((N,), jnp.float32),
            pltpu.VMEM((N,), jnp.float32),
            pltpu.VMEM((M_PAD,), jnp.float32),
            pltpu.VMEM((M_PAD,), jnp.float32),
            pltpu.VMEM((M_PAD,), jnp.float32),
            pltpu.VMEM((M_PAD,), jnp.float32),
            pltpu.VMEM((FPT, M_PAD), jnp.float32),
            pltpu.VMEM((CHUNK,), jnp.int32),
            pltpu.VMEM((CHUNK,), jnp.int32),
        ],
    )(_sc_body)
    return kfn(uT, cT, src, dst)


def kernel(x, pos, out_pos, edge_index, W1, b1):
    u = _compute_u(x, pos, W1)
    c = _compute_c(out_pos, W1, b1)
    uT = u.T
    cT = jnp.pad(c, ((0, M_PAD - M), (0, 0))).T
    src = edge_index[0]
    dst = edge_index[1]
    outT = _sc_scatter_max(uT, cT, src, dst)
    agg = outT[:, :M].T
    return (agg, out_pos)


# speedup vs baseline: 13.3739x; 4.7131x over previous
import functools

import jax
import jax.numpy as jnp
from jax import lax
from jax.experimental import pallas as pl
from jax.experimental.pallas import tpu as pltpu
from jax.experimental.pallas import tpu_sc as plsc

N = 10000
M = 5000
E = 320000
D = 128

M_PAD = 5024
M_TRASH = 5008
NTILES = 32
EPW = E // NTILES
CPB = 1024
ICH = 16 * CPB

NEG_PACK = -8323200


def _u_body(x_ref, pos_ref, w_ref, u_ref):
    xw = jnp.dot(x_ref[...], w_ref[:D, :], preferred_element_type=jnp.float32)
    pw = jnp.dot(pos_ref[...], w_ref[D:, :], preferred_element_type=jnp.float32)
    u_ref[...] = xw + pw


def _ep_body(s_ref, op_ref, w_ref, b_ref, o_ref):
    pw = jnp.dot(op_ref[...], w_ref[D:, :], preferred_element_type=jnp.float32)
    o_ref[...] = jnp.maximum(s_ref[...] + (b_ref[...][None, :] - pw), 0.0)


def _compute_u(x, pos, W1):
    bn = 1000
    return pl.pallas_call(
        _u_body,
        grid=(N // bn,),
        in_specs=[
            pl.BlockSpec((bn, D), lambda i: (i, 0)),
            pl.BlockSpec((bn, 3), lambda i: (i, 0)),
            pl.BlockSpec((D + 3, D), lambda i: (0, 0)),
        ],
        out_specs=pl.BlockSpec((bn, D), lambda i: (i, 0)),
        out_shape=jax.ShapeDtypeStruct((N, D), jnp.float32),
    )(x, pos, W1)


def _epilogue(s, out_pos, W1, b1):
    bm = 1000
    return pl.pallas_call(
        _ep_body,
        grid=(M // bm,),
        in_specs=[
            pl.BlockSpec((bm, D), lambda i: (i, 0)),
            pl.BlockSpec((bm, 3), lambda i: (i, 0)),
            pl.BlockSpec((D + 3, D), lambda i: (0, 0)),
            pl.BlockSpec((D,), lambda i: (0,)),
        ],
        out_specs=pl.BlockSpec((bm, D), lambda i: (i, 0)),
        out_shape=jax.ShapeDtypeStruct((M, D), jnp.float32),
    )(s, out_pos, W1, b1)


def _scan_count_base():
    cz, _ = plsc.scan_count(jnp.zeros((16,), jnp.int32))
    return jnp.min(cz)


def _bin_body(src_hbm, dst_hbm, bedge_hbm, ibedge_hbm, counts_hbm,
              sbuf, dbuf, oedge, oint, cnt16, cur16, base16):
    wid = lax.axis_index("s") * 2 + lax.axis_index("c")
    base = wid * EPW
    pltpu.sync_copy(src_hbm.at[pl.ds(base, EPW)], sbuf)
    pltpu.sync_copy(dst_hbm.at[pl.ds(base, EPW)], dbuf)

    c0 = _scan_count_base()
    cnt16[...] = jnp.zeros((16,), jnp.int32)

    def count_body(i, _):
        dv = dbuf[pl.ds(i * 16, 16)]
        b = jnp.bitwise_and(dv, 15)
        cntv, lastv = plsc.scan_count(b)
        plsc.addupdate_scatter(cnt16, [b], cntv - c0 + 1, mask=lastv)
        return 0

    lax.fori_loop(0, EPW // 16, count_body, 0)

    counts = cnt16[...]
    bases = plsc.cumsum(counts) - counts
    cur16[...] = bases
    base16[...] = bases

    lanes = lax.broadcasted_iota(jnp.int32, (16,), 0)
    sentinel = M_TRASH + lanes

    def fill_body(i, _):
        oint[pl.ds(i * 16, 16)] = sentinel
        return 0

    lax.fori_loop(0, CPB, fill_body, 0)

    def scatter_body(i, _):
        sl = pl.ds(i * 16, 16)
        sv = sbuf[sl]
        dv = dbuf[sl]
        b = jnp.bitwise_and(dv, 15)
        cntv, lastv = plsc.scan_count(b)
        ew = jnp.left_shift(sv, 13) | dv
        pos = plsc.load_gather(cur16, [b]) + (cntv - c0)
        plsc.store_scatter(oedge, [pos], ew)
        o = pos - plsc.load_gather(base16, [b])
        plsc.store_scatter(oint, [jnp.left_shift(o, 4) | b], ew,
                           mask=o < CPB)
        plsc.addupdate_scatter(cur16, [b], cntv - c0 + 1, mask=lastv)
        return 0

    lax.fori_loop(0, EPW // 16, scatter_body, 0)

    pltpu.sync_copy(oedge, bedge_hbm.at[pl.ds(base, EPW)])
    pltpu.sync_copy(oint, ibedge_hbm.at[pl.ds(wid * ICH, ICH)])
    pltpu.sync_copy(cnt16, counts_hbm.at[wid])


def _bin_edges(src, dst):
    mesh = plsc.VectorSubcoreMesh(core_axis_name="c", subcore_axis_name="s")
    kfn = functools.partial(
        pl.kernel,
        mesh=mesh,
        compiler_params=pltpu.CompilerParams(needs_layout_passes=False),
        out_type=(
            jax.ShapeDtypeStruct((E,), jnp.int32),
            jax.ShapeDtypeStruct((NTILES * ICH,), jnp.int32),
            jax.ShapeDtypeStruct((NTILES, 16), jnp.int32),
        ),
        scratch_types=[
            pltpu.VMEM((EPW,), jnp.int32),
            pltpu.VMEM((EPW,), jnp.int32),
            pltpu.VMEM((EPW,), jnp.int32),
            pltpu.VMEM((ICH,), jnp.int32),
            pltpu.VMEM((16,), jnp.int32),
            pltpu.VMEM((16,), jnp.int32),
            pltpu.VMEM((16,), jnp.int32),
        ],
    )(_bin_body)
    return kfn(src, dst)


def _sc_body(upk, bedge_hbm, ibedge_hbm, counts_hbm, outT,
             u01, u23, a01, a23, cnts,
             ebuf0, ebuf1, ibuf0, ibuf1, es0, es1, is0, is1):
    us = (u01, u23)
    accs = (a01, a23)
    ebufs = (ebuf0, ebuf1)
    ibufs = (ibuf0, ibuf1)
    esems = (es0, es1)
    isems = (is0, is1)
    wid = lax.axis_index("s") * 2 + lax.axis_index("c")
    row0 = wid * 2

    pltpu.sync_copy(upk.at[row0], u01)
    pltpu.sync_copy(upk.at[row0 + 1], u23)
    pltpu.sync_copy(counts_hbm, cnts)

    minus_inf = jnp.full((16,), NEG_PACK, jnp.int32)

    def init_body(i, _):
        for a_j in accs:
            a_j[pl.ds(i * 16, 16)] = minus_inf
        return 0

    lax.fori_loop(0, M_PAD // 16, init_body, 0)

    def issue(k, b):
        pltpu.make_async_copy(bedge_hbm.at[pl.ds(k * EPW, EPW)],
                              ebufs[b], esems[b]).start()
        pltpu.make_async_copy(ibedge_hbm.at[pl.ds(k * ICH, ICH)],
                              ibufs[b].at[pl.ds(0, ICH)], isems[b]).start()

    def wait(k, b):
        pltpu.make_async_copy(bedge_hbm.at[pl.ds(k * EPW, EPW)],
                              ebufs[b], esems[b]).wait()
        pltpu.make_async_copy(ibedge_hbm.at[pl.ds(k * ICH, ICH)],
                              ibufs[b].at[pl.ds(0, ICH)], isems[b]).wait()

    lanes = lax.broadcasted_iota(jnp.int32, (16,), 0)

    def rmw(sv, dv, valid):
        gs = [plsc.load_gather(us[j], [sv], mask=valid) for j in range(2)]
        os_ = [plsc.load_gather(accs[j], [dv], mask=valid)
               for j in range(2)]
        ms = [plsc.bitcast(
                  jnp.maximum(plsc.bitcast(g, jnp.bfloat16),
                              plsc.bitcast(o, jnp.bfloat16)),
                  jnp.int32)
              for g, o in zip(gs, os_)]
        for j in range(2):
            plsc.store_scatter(accs[j], [dv], ms[j], mask=valid)

    def process(k, ebuf, ibuf):
        cntv = plsc.load_gather(cnts, [k * 16 + lanes])
        maxc = jnp.max(cntv)

        @pl.when(maxc <= CPB)
        def _fast():
            def fetch(i):
                return ibuf[pl.ds(i * 16, 16)]

            def vec_body(i, ev):
                sv = jnp.right_shift(ev, 13)
                dv = jnp.bitwise_and(ev, 8191)
                evb = fetch(2 * i + 1)
                rmw(sv, dv, None)
                svb = jnp.right_shift(evb, 13)
                dvb = jnp.bitwise_and(evb, 8191)
                nxt = fetch(2 * i + 2)
                rmw(svb, dvb, None)
                return nxt

            lax.fori_loop(0, (maxc + 1) // 2, vec_body, fetch(0))

        @pl.when(maxc > CPB)
        def _general():
            basev = plsc.cumsum(cntv) - cntv

            def fetch(i):
                ib = jnp.full((16,), 0, jnp.int32) + i
                valid = ib < cntv
                ev = plsc.load_gather(ebuf, [basev + ib], mask=valid)
                return ev, valid

            def vec_body(i, carry):
                ev, valid = carry
                sv = jnp.right_shift(ev, 13)
                dv = jnp.bitwise_and(ev, 8191)
                nxt = fetch(i + 1)
                rmw(sv, dv, valid)
                return nxt

            lax.fori_loop(0, maxc, vec_body, fetch(0))

    issue(0, 0)

    def outer(g, _):
        for b in range(2):
            k = 2 * g + b

            @pl.when(k + 1 < NTILES)
            def _():
                issue(k + 1, 1 - b)

            wait(k, b)
            process(k, ebufs[b], ibufs[b])
        return 0

    lax.fori_loop(0, NTILES // 2, outer, 0)

    pltpu.sync_copy(a01, outT.at[row0])
    pltpu.sync_copy(a23, outT.at[row0 + 1])


def _sc_scatter_max(upk, bedge, ibedge, counts):
    mesh = plsc.VectorSubcoreMesh(core_axis_name="c", subcore_axis_name="s")
    kfn = functools.partial(
        pl.kernel,
        mesh=mesh,
        compiler_params=pltpu.CompilerParams(needs_layout_passes=False),
        out_type=jax.ShapeDtypeStruct((D // 2, M_PAD), jnp.int32),
        scratch_types=[
            pltpu.VMEM((N,), jnp.int32),
            pltpu.VMEM((N,), jnp.int32),
            pltpu.VMEM((M_PAD,), jnp.int32),
            pltpu.VMEM((M_PAD,), jnp.int32),
            pltpu.VMEM((NTILES * 16,), jnp.int32),
            pltpu.VMEM((EPW,), jnp.int32),
            pltpu.VMEM((EPW,), jnp.int32),
            pltpu.VMEM((ICH + 16,), jnp.int32),
            pltpu.VMEM((ICH + 16,), jnp.int32),
            pltpu.SemaphoreType.DMA,
            pltpu.SemaphoreType.DMA,
            pltpu.SemaphoreType.DMA,
            pltpu.SemaphoreType.DMA,
        ],
    )(_sc_body)
    return kfn(upk, bedge, ibedge, counts)


def kernel(x, pos, out_pos, edge_index, W1, b1):
    u = _compute_u(x, pos, W1)
    upk = jax.lax.bitcast_convert_type(
        u.astype(jnp.bfloat16).reshape(N, D // 2, 2), jnp.int32)
    src = edge_index[0]
    dst = edge_index[1]
    bedge, ibedge, counts = _bin_edges(src, dst)
    outT = _sc_scatter_max(upk.T, bedge, ibedge, counts.reshape(-1))
    s = jax.lax.bitcast_convert_type(
        outT.T[:M], jnp.bfloat16).reshape(M, D).astype(jnp.float32)
    agg = _epilogue(s, out_pos, W1, b1)
    return (agg, out_pos)
